# baseline (speedup 1.0000x reference)
import functools
import math

import jax
import jax.numpy as jnp
from jax import lax
from jax.experimental import pallas as pl
from jax.experimental.pallas import tpu as pltpu
from jax.experimental.pallas import tpu_sc as plsc

DIM = 64
SCALE = math.sqrt(DIM)
NUM_CORES = 2
NUM_SUBCORES = 16
NUM_WORKERS = NUM_CORES * NUM_SUBCORES
LANES = 16

CHUNK = 320
GATHER_SUB = 64
NSUB = CHUNK // GATHER_SUB


def _emb_kernel(num_rows):
    b_per_w = num_rows // NUM_WORKERS
    n_chunks = b_per_w // CHUNK
    mesh = plsc.VectorSubcoreMesh(core_axis_name="c", subcore_axis_name="s")

    @functools.partial(
        pl.kernel,
        mesh=mesh,
        out_type=jax.ShapeDtypeStruct((num_rows // 2, 2 * DIM), jnp.float32),
        scratch_types=[
            pltpu.VMEM((CHUNK,), jnp.int32),
            pltpu.VMEM((CHUNK,), jnp.int32),
            pltpu.VMEM((CHUNK,), jnp.int32),
            pltpu.VMEM((CHUNK,), jnp.int32),
            pltpu.VMEM((CHUNK, 2 * DIM), jnp.float32),
            pltpu.VMEM((CHUNK, 2 * DIM), jnp.float32),
            pltpu.VMEM((CHUNK // 2, 2 * DIM), jnp.float32),
            pltpu.VMEM((CHUNK // 2, 2 * DIM), jnp.float32),
            pltpu.SemaphoreType.DMA,
            pltpu.SemaphoreType.DMA,
            pltpu.SemaphoreType.DMA,
            pltpu.SemaphoreType.DMA,
        ],
        compiler_params=pltpu.CompilerParams(use_tc_tiling_on_sc=True),
    )
    def body(idx_hbm, pairs_hbm, out_hbm,
             idx_a, idx_b, idxp_a, idxp_b, rows_a, rows_b, out_a, out_b,
             gsem_a, gsem_b, ssem_a, ssem_b):
        wid = lax.axis_index("s") * NUM_CORES + lax.axis_index("c")
        base = wid * b_per_w
        ibufs = (idx_a, idx_b)
        pbufs = (idxp_a, idxp_b)
        rbufs = (rows_a, rows_b)
        obufs = (out_a, out_b)
        gsems = (gsem_a, gsem_b)
        ssems = (ssem_a, ssem_b)

        def load_idx(c, b):
            off = pl.multiple_of(base + c * CHUNK, CHUNK)
            pltpu.sync_copy(idx_hbm.at[pl.ds(off, CHUNK)], ibufs[b])

            def shift(i, carry):
                sl = pl.ds(i * LANES, LANES)
                pbufs[b][sl] = ibufs[b][sl] >> 1
                return carry

            lax.fori_loop(0, CHUNK // LANES, shift, 0, unroll=4)

        def fire_gather(b):
            for j in range(NSUB):
                sl = pl.ds(j * GATHER_SUB, GATHER_SUB)
                pltpu.async_copy(
                    pairs_hbm.at[pbufs[b].at[sl]],
                    rbufs[b].at[sl],
                    gsems[b],
                )

        def drain_gather(b):
            for j in range(NSUB):
                sl = pl.ds(j * GATHER_SUB, GATHER_SUB)
                pltpu.make_async_copy(
                    pairs_hbm.at[pbufs[b].at[sl]],
                    rbufs[b].at[sl],
                    gsems[b],
                ).wait()

        def wait_store(b):
            pltpu.make_async_copy(
                obufs[b], out_hbm.at[pl.ds(0, CHUNK // 2)], ssems[b]
            ).wait()

        load_idx(0, 0)
        fire_gather(0)

        def step(c, b):
            rows, outb = rbufs[b], obufs[b]

            @pl.when(c >= 2)
            def _():
                wait_store(b)

            @pl.when(c + 1 < n_chunks)
            def _():
                load_idx(c + 1, 1 - b)
                fire_gather(1 - b)

            drain_gather(b)

            def select_group(q, carry):
                v16 = ibufs[b][pl.ds(q * LANES, LANES)]
                for j in range(LANES):
                    t = q * LANES + j
                    odd = (v16[j] & 1) == 1
                    k = q * (LANES // 2) + j // 2
                    for g in range(DIM // LANES):
                        lo = rows[t, pl.ds(g * LANES, LANES)]
                        hi = rows[t, pl.ds(DIM + g * LANES, LANES)]
                        outb[k, pl.ds((j % 2) * DIM + g * LANES, LANES)] = (
                            jnp.where(odd, hi, lo) * SCALE
                        )
                return carry

            lax.fori_loop(0, CHUNK // LANES, select_group, 0)

            ooff = pl.multiple_of((base + c * CHUNK) // 2, CHUNK // 2)
            pltpu.async_copy(
                outb,
                out_hbm.at[pl.ds(ooff, CHUNK // 2)],
                ssems[b],
            )

        def pair_steps(p, carry):
            step(2 * p, 0)
            step(2 * p + 1, 1)
            return carry

        lax.fori_loop(0, n_chunks // 2, pair_steps, 0)
        wait_store((n_chunks - 1) % 2)

    return body


def kernel(x, table):
    num_rows = x.size
    idx = jnp.reshape(x, (num_rows,)).astype(jnp.int32)
    pairs = jnp.reshape(table, (table.shape[0] // 2, 2 * DIM))
    out = _emb_kernel(num_rows)(idx, pairs)
    return jnp.reshape(out, x.shape + (DIM,))


# speedup vs baseline: 1.1052x; 1.1052x over previous
import functools
import math

import jax
import jax.numpy as jnp
from jax import lax
from jax.experimental import pallas as pl
from jax.experimental.pallas import tpu as pltpu
from jax.experimental.pallas import tpu_sc as plsc

DIM = 64
SCALE = math.sqrt(DIM)
NUM_CORES = 2
NUM_SUBCORES = 16
NUM_WORKERS = NUM_CORES * NUM_SUBCORES
LANES = 16

SEQ = 200
GATHER_SUBS = (64, 64, 64, 8)


def _emb_kernel(num_seqs):
    s_per_w = num_seqs // NUM_WORKERS
    mesh = plsc.VectorSubcoreMesh(core_axis_name="c", subcore_axis_name="s")

    @functools.partial(
        pl.kernel,
        mesh=mesh,
        out_type=jax.ShapeDtypeStruct((num_seqs, SEQ, DIM), jnp.float32),
        scratch_types=[
            pltpu.VMEM((SEQ,), jnp.int32),
            pltpu.VMEM((SEQ,), jnp.int32),
            pltpu.VMEM((SEQ,), jnp.int32),
            pltpu.VMEM((SEQ,), jnp.int32),
            pltpu.VMEM((SEQ, 2 * DIM), jnp.float32),
            pltpu.VMEM((SEQ, 2 * DIM), jnp.float32),
            pltpu.VMEM((SEQ, DIM), jnp.float32),
            pltpu.VMEM((SEQ, DIM), jnp.float32),
            pltpu.SemaphoreType.DMA,
            pltpu.SemaphoreType.DMA,
            pltpu.SemaphoreType.DMA,
            pltpu.SemaphoreType.DMA,
        ],
        compiler_params=pltpu.CompilerParams(use_tc_tiling_on_sc=True),
    )
    def body(idx_hbm, pairs_hbm, out_hbm,
             idx_a, idx_b, idxp_a, idxp_b, rows_a, rows_b, out_a, out_b,
             gsem_a, gsem_b, ssem_a, ssem_b):
        wid = lax.axis_index("s") * NUM_CORES + lax.axis_index("c")
        sbase = wid * s_per_w
        ibufs = (idx_a, idx_b)
        pbufs = (idxp_a, idxp_b)
        rbufs = (rows_a, rows_b)
        obufs = (out_a, out_b)
        gsems = (gsem_a, gsem_b)
        ssems = (ssem_a, ssem_b)

        def load_idx(c, b):
            off = pl.multiple_of((sbase + c) * SEQ, 8)
            pltpu.sync_copy(idx_hbm.at[pl.ds(off, SEQ)], ibufs[b])

            def shift(i, carry):
                sl = pl.ds(i * LANES, LANES)
                pbufs[b][sl] = ibufs[b][sl] >> 1
                return carry

            lax.fori_loop(0, SEQ // LANES, shift, 0, unroll=4)
            sl = pl.ds(SEQ - LANES, LANES)
            pbufs[b][sl] = ibufs[b][sl] >> 1

        def fire_gather(b):
            off = 0
            for n in GATHER_SUBS:
                sl = pl.ds(off, n)
                pltpu.async_copy(
                    pairs_hbm.at[pbufs[b].at[sl]],
                    rbufs[b].at[sl],
                    gsems[b],
                )
                off += n

        def drain_gather(b):
            off = 0
            for n in GATHER_SUBS:
                sl = pl.ds(off, n)
                pltpu.make_async_copy(
                    pairs_hbm.at[pbufs[b].at[sl]],
                    rbufs[b].at[sl],
                    gsems[b],
                ).wait()
                off += n

        def wait_store(b):
            pltpu.make_async_copy(
                obufs[b], out_hbm.at[sbase], ssems[b]
            ).wait()

        load_idx(0, 0)
        fire_gather(0)

        def step(c, b):
            rows, outb = rbufs[b], obufs[b]

            @pl.when(c >= 2)
            def _():
                wait_store(b)

            @pl.when(c + 1 < s_per_w)
            def _():
                load_idx(c + 1, 1 - b)
                fire_gather(1 - b)

            drain_gather(b)

            def select_16(tbase):
                v16 = ibufs[b][pl.ds(tbase, LANES)]
                for j in range(LANES):
                    t = tbase + j
                    odd = (v16[j] & 1) == 1
                    for g in range(DIM // LANES):
                        lo = rows[t, pl.ds(g * LANES, LANES)]
                        hi = rows[t, pl.ds(DIM + g * LANES, LANES)]
                        outb[t, pl.ds(g * LANES, LANES)] = (
                            jnp.where(odd, hi, lo) * SCALE
                        )

            def select_group(q, carry):
                select_16(q * LANES)
                return carry

            lax.fori_loop(0, SEQ // LANES, select_group, 0)
            select_16(SEQ - LANES)

            pltpu.async_copy(outb, out_hbm.at[sbase + c], ssems[b])

        def pair_steps(p, carry):
            step(2 * p, 0)
            step(2 * p + 1, 1)
            return carry

        lax.fori_loop(0, s_per_w // 2, pair_steps, 0)
        wait_store((s_per_w - 1) % 2)

    return body


def kernel(x, table):
    num_seqs = x.shape[0]
    idx = jnp.reshape(x, (x.size,)).astype(jnp.int32)
    pairs = jnp.reshape(table, (table.shape[0] // 2, 2 * DIM))
    return _emb_kernel(num_seqs)(idx, pairs)
